# baseline (speedup 1.0000x reference)
import functools

import jax
import jax.numpy as jnp
from jax import lax
from jax.experimental import pallas as pl
from jax.experimental.pallas import tpu as pltpu
from jax.experimental.pallas import tpu_sc as plsc

N_LAYERS = 5
N = 10000
E = 320000
EMB = 200
EMBP = 256
NC, NS = 2, 16
NW = NC * NS
CH = 80
CPT = (E // NW) // CH
EPT = E // NW


@functools.cache
def _msg_kernel():
  mesh = plsc.VectorSubcoreMesh(
      core_axis_name="c", subcore_axis_name="s", num_cores=NC, num_subcores=NS)

  @functools.partial(
      pl.kernel,
      out_type=jax.ShapeDtypeStruct((E, EMBP), jnp.float32),
      mesh=mesh,
      scratch_types=[
          pltpu.VMEM((CPT, CH), jnp.int32),
          pltpu.VMEM((CH, EMBP), jnp.float32),
          pltpu.VMEM((CH, EMBP), jnp.float32),
          pltpu.SemaphoreType.DMA,
      ],
  )
  def k(h_hbm, ee_hbm, src_hbm, msg_hbm, idx_v, buf, gbuf, sem):
    c = lax.axis_index("c")
    s = lax.axis_index("s")
    wid = c * NS + s
    base_w = wid * EPT
    pltpu.sync_copy(src_hbm.at[wid], idx_v)

    def body(j, carry):
      base = base_w + j * CH
      pltpu.sync_copy(ee_hbm.at[pl.ds(base, CH)], buf)
      pltpu.async_copy(h_hbm.at[idx_v.at[j]], gbuf, sem).wait()

      def row(r, carry2):
        for cc in range(13):
          sl = pl.ds(cc * 16, 16)
          buf[r, sl] = buf[r, sl] + gbuf[r, sl]
        return carry2

      lax.fori_loop(0, CH, row, 0)
      pltpu.sync_copy(buf, msg_hbm.at[pl.ds(base, CH)])
      return carry

    lax.fori_loop(0, CPT, body, 0)

  return k


def _msg_sc(h, ee, src3):
  return _msg_kernel()(h, ee, src3)


def _mm_body(x_ref, w_ref, b_ref, o_ref):
  o_ref[...] = jnp.dot(x_ref[...], w_ref[...],
                       preferred_element_type=jnp.float32) + b_ref[...]


def _mm(x, w, b, bm):
  m, k = x.shape
  n = w.shape[1]
  g = m // bm
  return pl.pallas_call(
      _mm_body,
      grid=(g,),
      in_specs=[
          pl.BlockSpec((bm, k), lambda i: (i, 0)),
          pl.BlockSpec((k, n), lambda i: (0, 0)),
          pl.BlockSpec((1, n), lambda i: (0, 0)),
      ],
      out_specs=pl.BlockSpec((bm, n), lambda i: (i, 0)),
      out_shape=jax.ShapeDtypeStruct((m, n), jnp.float32),
  )(x, w, b.reshape(1, n))


def kernel(node_feat, edge_feat, edge_index, node_W, node_b, edge_W, edge_b,
           mlp_W1, mlp_b1, mlp_W2, mlp_b2, bn_gamma, bn_beta,
           out_W1, out_b1, out_W2, out_b2):
  src = edge_index[0]
  dst = edge_index[1]
  n = node_feat.shape[0]
  src3 = src.astype(jnp.int32).reshape(NW, CPT, CH)
  h = _mm(node_feat, node_W, node_b, 2000)
  eW = jnp.pad(edge_W, ((0, 0), (0, 0), (0, EMBP - EMB)))
  eb = jnp.pad(edge_b, ((0, 0), (0, EMBP - EMB)))
  for l in range(N_LAYERS):
    ee256 = _mm(edge_feat, eW[l], eb[l], 8000)
    h_p = jnp.pad(h, ((0, 0), (0, EMBP - EMB)))
    msg256 = lax.optimization_barrier(
        jnp.pad(h[src] + ee256[:, :200], ((0, 0), (0, 56))))
    agg = jax.ops.segment_sum(msg256, dst, num_segments=n)[:, :200]
    hidden = jnp.maximum(agg @ mlp_W1[l] + mlp_b1[l], 0.0)
    h2 = hidden @ mlp_W2[l] + mlp_b2[l]
    mean = jnp.mean(h2, axis=0)
    var = jnp.var(h2, axis=0)
    h2 = (h2 - mean) / jnp.sqrt(var + 1e-5) * bn_gamma[l] + bn_beta[l]
    if l < N_LAYERS - 1:
      h2 = jnp.maximum(h2, 0.0)
    h = h2
  pooled = jnp.sum(h, axis=0, keepdims=True)
  feats = jnp.maximum(pooled @ out_W1 + out_b1, 0.0) @ out_W2 + out_b2
  return feats


# speedup vs baseline: 1.1738x; 1.1738x over previous
import jax
import jax.numpy as jnp
from jax.experimental import pallas as pl

N_LAYERS = 5


def _mm_body(x_ref, w_ref, b_ref, o_ref):
  o_ref[...] = jnp.dot(x_ref[...], w_ref[...],
                       preferred_element_type=jnp.float32) + b_ref[...]


def _mm(x, w, b, bm):
  m, k = x.shape
  n = w.shape[1]
  g = m // bm
  return pl.pallas_call(
      _mm_body,
      grid=(g,),
      in_specs=[
          pl.BlockSpec((bm, k), lambda i: (i, 0)),
          pl.BlockSpec((k, n), lambda i: (0, 0)),
          pl.BlockSpec((1, n), lambda i: (0, 0)),
      ],
      out_specs=pl.BlockSpec((bm, n), lambda i: (i, 0)),
      out_shape=jax.ShapeDtypeStruct((m, n), jnp.float32),
  )(x, w, b.reshape(1, n))


def kernel(node_feat, edge_feat, edge_index, node_W, node_b, edge_W, edge_b,
           mlp_W1, mlp_b1, mlp_W2, mlp_b2, bn_gamma, bn_beta,
           out_W1, out_b1, out_W2, out_b2):
  src = edge_index[0]
  dst = edge_index[1]
  n = node_feat.shape[0]
  h = _mm(node_feat, node_W, node_b, 2000)
  for l in range(N_LAYERS):
    edge_embeds = _mm(edge_feat, edge_W[l], edge_b[l], 8000)
    msg = h[src] + edge_embeds
    agg = jax.ops.segment_sum(msg, dst, num_segments=n)
    hidden = jnp.maximum(agg @ mlp_W1[l] + mlp_b1[l], 0.0)
    h2 = hidden @ mlp_W2[l] + mlp_b2[l]
    mean = jnp.mean(h2, axis=0)
    var = jnp.var(h2, axis=0)
    h2 = (h2 - mean) / jnp.sqrt(var + 1e-5) * bn_gamma[l] + bn_beta[l]
    if l < N_LAYERS - 1:
      h2 = jnp.maximum(h2, 0.0)
    h = h2
  pooled = jnp.sum(h, axis=0, keepdims=True)
  feats = jnp.maximum(pooled @ out_W1 + out_b1, 0.0) @ out_W2 + out_b2
  return feats


# speedup vs baseline: 1.1758x; 1.0017x over previous
import jax
import jax.numpy as jnp
from jax.experimental import pallas as pl

N_LAYERS = 5


def _mm_body(x_ref, w_ref, b_ref, o_ref):
  o_ref[...] = jnp.dot(x_ref[...], w_ref[...],
                       preferred_element_type=jnp.float32) + b_ref[...]


def _mm(x, w, b, bm):
  m, k = x.shape
  n = w.shape[1]
  g = m // bm
  return pl.pallas_call(
      _mm_body,
      grid=(g,),
      in_specs=[
          pl.BlockSpec((bm, k), lambda i: (i, 0)),
          pl.BlockSpec((k, n), lambda i: (0, 0)),
          pl.BlockSpec((1, n), lambda i: (0, 0)),
      ],
      out_specs=pl.BlockSpec((bm, n), lambda i: (i, 0)),
      out_shape=jax.ShapeDtypeStruct((m, n), jnp.float32),
  )(x, w, b.reshape(1, n))


def kernel(node_feat, edge_feat, edge_index, node_W, node_b, edge_W, edge_b,
           mlp_W1, mlp_b1, mlp_W2, mlp_b2, bn_gamma, bn_beta,
           out_W1, out_b1, out_W2, out_b2):
  src = edge_index[0]
  dst = edge_index[1]
  n = node_feat.shape[0]
  h = _mm(node_feat, node_W, node_b, 10000)
  for l in range(N_LAYERS):
    edge_embeds = _mm(edge_feat, edge_W[l], edge_b[l], 16000)
    msg = h[src] + edge_embeds
    agg = jax.ops.segment_sum(msg, dst, num_segments=n)
    hidden = jnp.maximum(agg @ mlp_W1[l] + mlp_b1[l], 0.0)
    h2 = hidden @ mlp_W2[l] + mlp_b2[l]
    mean = jnp.mean(h2, axis=0)
    var = jnp.var(h2, axis=0)
    h2 = (h2 - mean) / jnp.sqrt(var + 1e-5) * bn_gamma[l] + bn_beta[l]
    if l < N_LAYERS - 1:
      h2 = jnp.maximum(h2, 0.0)
    h = h2
  pooled = jnp.sum(h, axis=0, keepdims=True)
  feats = jnp.maximum(pooled @ out_W1 + out_b1, 0.0) @ out_W2 + out_b2
  return feats
